# baseline (speedup 1.0000x reference)
import functools

import jax
import jax.numpy as jnp
from jax import lax
from jax.experimental import pallas as pl
from jax.experimental.pallas import tpu as pltpu
from jax.experimental.pallas import tpu_sc as plsc

_LANES = 16
_GW = 128


@functools.lru_cache(maxsize=None)
def _make_sc_lookup(n_tok: int, vocab: int, d: int, scale: float):
    info = plsc.get_sparse_core_info()
    nc, ns = info.num_cores, info.num_subcores
    nw = nc * ns
    assert d % _LANES == 0
    chunk = 512
    assert n_tok % (nw * chunk) == 0
    per_w = n_tok // nw
    n_chunks = per_w // chunk
    gpc = chunk // _GW
    rows_per_idx_row = _GW
    idx_rows_per_chunk = chunk // rows_per_idx_row

    mesh = plsc.VectorSubcoreMesh(core_axis_name="c", subcore_axis_name="s")

    @functools.partial(
        pl.kernel,
        out_type=jax.ShapeDtypeStruct((n_tok, d), jnp.float32),
        mesh=mesh,
        scratch_types=[
            pltpu.VMEM((idx_rows_per_chunk, _GW), jnp.int32),
            pltpu.VMEM((chunk, d), jnp.float32),
            pltpu.SemaphoreType.DMA,
        ],
        compiler_params=pltpu.CompilerParams(use_tc_tiling_on_sc=False),
    )
    def k(idx_hbm, table_hbm, out_hbm, idx_v, rows_v, sem):
        wid = lax.axis_index("s") * nc + lax.axis_index("c")
        idx_row0 = wid * (per_w // rows_per_idx_row)
        row0 = wid * per_w

        def chunk_body(g, carry):
            pltpu.sync_copy(
                idx_hbm.at[pl.ds(idx_row0 + g * idx_rows_per_chunk,
                                 idx_rows_per_chunk)],
                idx_v,
            )
            copies = [
                pltpu.async_copy(
                    table_hbm.at[idx_v.at[j]],
                    rows_v.at[pl.ds(j * _GW, _GW)],
                    sem,
                )
                for j in range(gpc)
            ]
            for c in copies:
                c.wait()

            def scale_body(r, carry2):
                for j in range(d // _LANES):
                    sl = pl.ds(j * _LANES, _LANES)
                    rows_v[r, sl] = rows_v[r, sl] * scale
                return carry2

            lax.fori_loop(0, chunk, scale_body, 0, unroll=4)

            pltpu.sync_copy(rows_v, out_hbm.at[pl.ds(row0 + g * chunk, chunk)])
            return carry

        lax.fori_loop(0, n_chunks, chunk_body, 0)

    return k


def kernel(input, table):
    vocab, d = table.shape
    n_tok = input.shape[0] * input.shape[1]
    scale = float(d) ** 0.5
    idx = input.reshape(n_tok // _GW, _GW).astype(jnp.int32)
    out = _make_sc_lookup(n_tok, vocab, d, scale)(idx, table)
    return out.reshape(*input.shape, d)


# speedup vs baseline: 1.0901x; 1.0901x over previous
import functools

import jax
import jax.numpy as jnp
from jax import lax
from jax.experimental import pallas as pl
from jax.experimental.pallas import tpu as pltpu
from jax.experimental.pallas import tpu_sc as plsc

_LANES = 16
_GW = 128
_CHUNK = 512


@functools.lru_cache(maxsize=None)
def _make_sc_lookup(n_tok: int, vocab: int, d: int, scale: float):
    info = plsc.get_sparse_core_info()
    nc, ns = info.num_cores, info.num_subcores
    nw = nc * ns
    assert d % _LANES == 0
    assert n_tok % (nw * _CHUNK) == 0
    per_w = n_tok // nw
    n_chunks = per_w // _CHUNK
    assert n_chunks % 2 == 0 and n_chunks >= 4
    gpc = _CHUNK // _GW
    idx_rpc = _CHUNK // _GW

    mesh = plsc.VectorSubcoreMesh(core_axis_name="c", subcore_axis_name="s")

    @functools.partial(
        pl.kernel,
        out_type=jax.ShapeDtypeStruct((n_tok, d), jnp.float32),
        mesh=mesh,
        scratch_types=[
            pltpu.VMEM((2, idx_rpc, _GW), jnp.int32),
            pltpu.VMEM((2, _CHUNK, d), jnp.float32),
            pltpu.SemaphoreType.DMA,
            pltpu.SemaphoreType.DMA,
            pltpu.SemaphoreType.DMA,
            pltpu.SemaphoreType.DMA,
        ],
        compiler_params=pltpu.CompilerParams(use_tc_tiling_on_sc=False),
    )
    def k(idx_hbm, table_hbm, out_hbm, idx_v, rows_v, g0, g1, w0, w1):
        gsem = (g0, g1)
        wsem = (w0, w1)
        wid = lax.axis_index("s") * nc + lax.axis_index("c")
        idx_row0 = wid * (per_w // _GW)
        row0 = wid * per_w

        def load_idx(b, g):
            pltpu.sync_copy(
                idx_hbm.at[pl.ds(idx_row0 + g * idx_rpc, idx_rpc)],
                idx_v.at[b],
            )

        def fire(b, g):
            for j in range(gpc):
                pltpu.async_copy(
                    table_hbm.at[idx_v.at[b, j]],
                    rows_v.at[b, pl.ds(j * _GW, _GW)],
                    gsem[b],
                )

        def drain(b):
            for j in range(gpc):
                pltpu.make_async_copy(
                    table_hbm.at[idx_v.at[b, j]],
                    rows_v.at[b, pl.ds(j * _GW, _GW)],
                    gsem[b],
                ).wait()

        def scale_buf(b):
            def body(r, carry):
                for j in range(d // _LANES):
                    sl = pl.ds(j * _LANES, _LANES)
                    rows_v[b, r, sl] = rows_v[b, r, sl] * scale
                return carry

            lax.fori_loop(0, _CHUNK, body, 0, unroll=8)

        def write(b, g):
            pltpu.async_copy(
                rows_v.at[b],
                out_hbm.at[pl.ds(row0 + g * _CHUNK, _CHUNK)],
                wsem[b],
            )

        def wait_write(b, g):
            pltpu.make_async_copy(
                rows_v.at[b],
                out_hbm.at[pl.ds(row0 + g * _CHUNK, _CHUNK)],
                wsem[b],
            ).wait()

        load_idx(0, 0)
        fire(0, 0)
        load_idx(1, 1)
        fire(1, 1)

        def pair_body(i, carry):
            for b in range(2):
                g = 2 * i + b
                drain(b)
                scale_buf(b)
                write(b, g)

                @pl.when(g + 2 < n_chunks)
                def _prefetch():
                    load_idx(b, g + 2)
                    wait_write(b, g)
                    fire(b, g + 2)

                @pl.when(g + 2 >= n_chunks)
                def _final_drain():
                    wait_write(b, g)

            return carry

        lax.fori_loop(0, n_chunks // 2, pair_body, 0)

    return k


def kernel(input, table):
    vocab, d = table.shape
    n_tok = input.shape[0] * input.shape[1]
    scale = float(d) ** 0.5
    idx = input.reshape(n_tok // _GW, _GW).astype(jnp.int32)
    out = _make_sc_lookup(n_tok, vocab, d, scale)(idx, table)
    return out.reshape(*input.shape, d)
